# baseline (speedup 1.0000x reference)
import functools

import jax
import jax.numpy as jnp
from jax import lax
from jax.experimental import pallas as pl
from jax.experimental.pallas import tpu as pltpu
from jax.experimental.pallas import tpu_sc as plsc

EMBED_DIM = 32
BATCH = 16384
HIST = 20
B_FLAT = BATCH * HIST

NUM_CORES = 2
NUM_SUBCORES = 16
NUM_WORKERS = NUM_CORES * NUM_SUBCORES
B_PER_W = BATCH // NUM_WORKERS
JCHUNKS = B_PER_W // 128
NSTEPS = HIST * JCHUNKS
NTILES = HIST * 4 * 128

_mesh = plsc.VectorSubcoreMesh(
    core_axis_name="c", subcore_axis_name="s",
    num_cores=NUM_CORES, num_subcores=NUM_SUBCORES,
)


ROW_PAD = 128


@functools.partial(
    pl.kernel,
    out_type=jax.ShapeDtypeStruct((NTILES, 1024), jnp.float32),
    mesh=_mesh,
    compiler_params=pltpu.CompilerParams(
        use_tc_tiling_on_sc=False, needs_layout_passes=False
    ),
    scratch_types=(
        [pltpu.VMEM((HIST * B_PER_W,), jnp.int32)]
        + [pltpu.VMEM((128, ROW_PAD), jnp.float32) for _ in range(2)]
        + [pltpu.VMEM((4 * 1024,), jnp.float32) for _ in range(2)]
        + [pltpu.SemaphoreType.DMA for _ in range(4)]
    ),
)
def _gather_kernel(table_hbm, idx_hbm, out_hbm, idx_v,
                   rows0, rows1, btile0, btile1, g0, g1, w0, w1):
    rows = (rows0, rows1)
    btile = (btile0, btile1)
    gsem = (g0, g1)
    wsem = (w0, w1)
    wid = lax.axis_index("s") * NUM_CORES + lax.axis_index("c")
    pltpu.sync_copy(idx_hbm.at[wid], idx_v)

    iota = lax.iota(jnp.int32, 16)
    dbvecs = [iota + 16 * k for k in range(8)]
    colvs = [lax.rem(iota + c, 32) for c in range(EMBED_DIM)]
    colv128 = [v * 128 for v in colvs]

    def gather_start(t, p):
        h = lax.rem(t, HIST)
        j = lax.div(t, HIST)
        off = h * B_PER_W + j * 128
        return pltpu.async_copy(
            table_hbm.at[idx_v.at[pl.ds(off, 128)]], rows[p], gsem[p]
        )

    gather_start(0, 0)
    gather_start(1, 1)

    def step(tt, carry):
        for p in (0, 1):
            t = 2 * tt + p
            h = lax.rem(t, HIST)
            j = lax.div(t, HIST)
            @pl.when(tt > 0)
            def _():
                for i in range(4):
                    pltpu.make_async_copy(
                        out_hbm.at[0], btile[p].at[pl.ds(i * 1024, 1024)],
                        wsem[p],
                    ).wait()
            pltpu.make_async_copy(
                table_hbm.at[pl.ds(0, 128)], rows[p], gsem[p]
            ).wait()
            for c2 in range(16):
                vals = [
                    plsc.load_gather(rows[p], [dbvecs[k], colvs[c]])
                    for c in (2 * c2, 2 * c2 + 1)
                    for k in range(8)
                ]
                for ci, c in enumerate((2 * c2, 2 * c2 + 1)):
                    for k in range(8):
                        plsc.store_scatter(
                            btile[p], [colv128[c] + dbvecs[k]],
                            vals[8 * ci + k],
                        )
            base = h * 512 + wid * JCHUNKS + j
            for i in range(4):
                pltpu.async_copy(
                    btile[p].at[pl.ds(i * 1024, 1024)],
                    out_hbm.at[base + i * 128],
                    wsem[p],
                )
            @pl.when(tt < (NSTEPS // 2) - 1)
            def _():
                gather_start(t + 2, p)
        return carry

    lax.fori_loop(0, NSTEPS // 2, step, 0)
    for p in (0, 1):
        for i in range(4):
            pltpu.make_async_copy(
                out_hbm.at[0], btile[p].at[pl.ds(i * 1024, 1024)], wsem[p]
            ).wait()


def kernel(input_ids, table):
    idx = (
        input_ids.reshape(NUM_WORKERS, B_PER_W, HIST)
        .transpose(0, 2, 1)
        .reshape(NUM_WORKERS, HIST * B_PER_W)
        .astype(jnp.int32)
    )
    tbl128 = jnp.pad(table, ((0, 0), (0, ROW_PAD - EMBED_DIM)))
    view = _gather_kernel(tbl128, idx)
    t = view.reshape(HIST, 4, 128, 8, 128)
    return t.transpose(2, 4, 0, 1, 3).reshape(BATCH, HIST, EMBED_DIM)


# speedup vs baseline: 1.0149x; 1.0149x over previous
import functools

import jax
import jax.numpy as jnp
from jax import lax
from jax.experimental import pallas as pl
from jax.experimental.pallas import tpu as pltpu
from jax.experimental.pallas import tpu_sc as plsc

EMBED_DIM = 32
BATCH = 16384
HIST = 20
B_FLAT = BATCH * HIST

NUM_CORES = 2
NUM_SUBCORES = 16
NUM_WORKERS = NUM_CORES * NUM_SUBCORES
B_PER_W = BATCH // NUM_WORKERS
JCHUNKS = B_PER_W // 128
NSTEPS = HIST * JCHUNKS
NTILES = HIST * 4 * 128

_mesh = plsc.VectorSubcoreMesh(
    core_axis_name="c", subcore_axis_name="s",
    num_cores=NUM_CORES, num_subcores=NUM_SUBCORES,
)


@functools.partial(
    pl.kernel,
    out_type=jax.ShapeDtypeStruct((NTILES, 1024), jnp.float32),
    mesh=_mesh,
    compiler_params=pltpu.CompilerParams(
        use_tc_tiling_on_sc=False, needs_layout_passes=False
    ),
    scratch_types=(
        [pltpu.VMEM((HIST * B_PER_W,), jnp.int32)]
        + [pltpu.VMEM((128, EMBED_DIM), jnp.float32) for _ in range(2)]
        + [pltpu.VMEM((4 * 1024,), jnp.float32) for _ in range(2)]
        + [pltpu.SemaphoreType.DMA for _ in range(4)]
    ),
)
def _gather_kernel(table_hbm, idx_hbm, out_hbm, idx_v,
                   rows0, rows1, btile0, btile1, g0, g1, w0, w1):
    rows = (rows0, rows1)
    btile = (btile0, btile1)
    gsem = (g0, g1)
    wsem = (w0, w1)
    wid = lax.axis_index("s") * NUM_CORES + lax.axis_index("c")
    pltpu.sync_copy(idx_hbm.at[wid], idx_v)

    iota = lax.iota(jnp.int32, 16)
    dbvecs = [iota + 16 * k for k in range(8)]
    colvs = [lax.rem(iota + c, 32) for c in range(EMBED_DIM)]
    colv128 = [v * 128 for v in colvs]

    def gather_start(t, p):
        h = lax.rem(t, HIST)
        j = lax.div(t, HIST)
        off = h * B_PER_W + j * 128
        return pltpu.async_copy(
            table_hbm.at[idx_v.at[pl.ds(off, 128)]], rows[p], gsem[p]
        )

    gather_start(0, 0)
    gather_start(1, 1)

    def step(tt, carry):
        for p in (0, 1):
            t = 2 * tt + p
            h = lax.rem(t, HIST)
            j = lax.div(t, HIST)
            @pl.when(tt > 0)
            def _():
                for i in range(4):
                    pltpu.make_async_copy(
                        out_hbm.at[0], btile[p].at[pl.ds(i * 1024, 1024)],
                        wsem[p],
                    ).wait()
            pltpu.make_async_copy(
                table_hbm.at[pl.ds(0, 128)], rows[p], gsem[p]
            ).wait()
            for c2 in range(16):
                vals = [
                    plsc.load_gather(rows[p], [dbvecs[k], colvs[c]])
                    for c in (2 * c2, 2 * c2 + 1)
                    for k in range(8)
                ]
                for ci, c in enumerate((2 * c2, 2 * c2 + 1)):
                    for k in range(8):
                        plsc.store_scatter(
                            btile[p], [colv128[c] + dbvecs[k]],
                            vals[8 * ci + k],
                        )
            base = h * 512 + wid * JCHUNKS + j
            for i in range(4):
                pltpu.async_copy(
                    btile[p].at[pl.ds(i * 1024, 1024)],
                    out_hbm.at[base + i * 128],
                    wsem[p],
                )
            @pl.when(tt < (NSTEPS // 2) - 1)
            def _():
                gather_start(t + 2, p)
        return carry

    lax.fori_loop(0, NSTEPS // 2, step, 0)
    for p in (0, 1):
        for i in range(4):
            pltpu.make_async_copy(
                out_hbm.at[0], btile[p].at[pl.ds(i * 1024, 1024)], wsem[p]
            ).wait()


def kernel(input_ids, table):
    idx = (
        input_ids.reshape(NUM_WORKERS, B_PER_W, HIST)
        .transpose(0, 2, 1)
        .reshape(NUM_WORKERS, HIST * B_PER_W)
        .astype(jnp.int32)
    )
    view = _gather_kernel(table, idx)
    t = view.reshape(HIST, 4, 128, 8, 128)
    return t.transpose(2, 4, 0, 1, 3).reshape(BATCH, HIST, EMBED_DIM)
